# baseline (speedup 1.0000x reference)
import jax
import jax.numpy as jnp
from jax.experimental import pallas as pl
from jax.experimental.pallas import tpu as pltpu

_N = 262144
_TOT = 4 * _N
_CH = 131072
_STEPS = _TOT // _CH


def _reg_kernel(td_ref, od_ref, out_ref, acc_ref):
    i = pl.program_id(0)

    @pl.when(i == 0)
    def _init():
        acc_ref[0] = 0.0

    td = jnp.reshape(td_ref[...], (_CH // 128, 128))
    od = jnp.reshape(od_ref[...], (_CH // 128, 128))
    diff = jnp.abs(od - td)
    sl1 = jnp.where(diff < 1.0, 0.5 * diff * diff, diff - 0.5)
    acc_ref[0] += jnp.sum(sl1)

    @pl.when(i == _STEPS - 1)
    def _finalize():
        out_ref[0, 0] = acc_ref[0]


def kernel(target_deltas, target_scores, output_deltas, output_scores):
    td = target_deltas.reshape(_TOT)
    od = output_deltas.reshape(_TOT)
    out = pl.pallas_call(
        _reg_kernel,
        grid=(_STEPS,),
        in_specs=[
            pl.BlockSpec((_CH,), lambda i: (i,)),
            pl.BlockSpec((_CH,), lambda i: (i,)),
        ],
        out_specs=pl.BlockSpec((1, 1), lambda i: (0, 0), memory_space=pltpu.SMEM),
        out_shape=jax.ShapeDtypeStruct((1, 1), jnp.float32),
        scratch_shapes=[pltpu.SMEM((1,), jnp.float32)],
        compiler_params=pltpu.CompilerParams(
            dimension_semantics=("arbitrary",),
        ),
    )(td, od)
    return out[0, 0]


# speedup vs baseline: 27.6554x; 27.6554x over previous
import jax
import jax.numpy as jnp
from jax.experimental import pallas as pl
from jax.experimental.pallas import tpu as pltpu

_N = 262144
_EPS = 1e-7
_ROWS = _N // 128
_BLK = 256
_STEPS = _ROWS // _BLK


def _rpn_loss_kernel(ts_ref, os_ref, td_ref, od_ref, out_ref, acc_ref):
    i = pl.program_id(0)

    @pl.when(i == 0)
    def _init():
        acc_ref[0] = 0.0
        acc_ref[1] = 0.0
        acc_ref[2] = 0.0
        acc_ref[3] = 0.0

    ts = ts_ref[...]
    osc = os_ref[...]
    valid = (ts != -1.0).astype(jnp.float32)
    o = jnp.clip(osc, _EPS, 1.0 - _EPS)
    bce = -(ts * jnp.log(o) + (1.0 - ts) * jnp.log(1.0 - o))
    p_star = (ts > 0.0).astype(jnp.float32)

    diff = jnp.abs(od_ref[...] - td_ref[...])
    sl1 = jnp.where(diff < 1.0, 0.5 * diff * diff, diff - 0.5)
    a_y = jnp.sum(sl1, axis=1)

    acc_ref[0] += jnp.sum(bce * valid)
    acc_ref[1] += jnp.sum(valid)
    acc_ref[2] += jnp.sum(a_y * p_star)
    acc_ref[3] += jnp.sum(p_star)

    @pl.when(i == _STEPS - 1)
    def _finalize():
        cls_loss = acc_ref[0] / jnp.maximum(acc_ref[1], 1.0)
        reg_loss = 10.0 * acc_ref[2] / jnp.maximum(_EPS, acc_ref[3])
        out_ref[0, 0] = cls_loss + reg_loss


def _planar(deltas):
    return jnp.transpose(deltas.reshape(_ROWS, 128, 4), (0, 2, 1))


def kernel(target_deltas, target_scores, output_deltas, output_scores):
    ts = target_scores.reshape(_ROWS, 128)
    osc = output_scores.reshape(_ROWS, 128)
    td = _planar(target_deltas)
    od = _planar(output_deltas)

    out = pl.pallas_call(
        _rpn_loss_kernel,
        grid=(_STEPS,),
        in_specs=[
            pl.BlockSpec((_BLK, 128), lambda i: (i, 0)),
            pl.BlockSpec((_BLK, 128), lambda i: (i, 0)),
            pl.BlockSpec((_BLK, 4, 128), lambda i: (i, 0, 0)),
            pl.BlockSpec((_BLK, 4, 128), lambda i: (i, 0, 0)),
        ],
        out_specs=pl.BlockSpec((1, 1), lambda i: (0, 0), memory_space=pltpu.SMEM),
        out_shape=jax.ShapeDtypeStruct((1, 1), jnp.float32),
        scratch_shapes=[pltpu.SMEM((4,), jnp.float32)],
        compiler_params=pltpu.CompilerParams(
            dimension_semantics=("arbitrary",),
        ),
    )(ts, osc, td, od)
    return out[0, 0]


# speedup vs baseline: 28.6879x; 1.0373x over previous
import jax
import jax.numpy as jnp
from jax.experimental import pallas as pl
from jax.experimental.pallas import tpu as pltpu

_N = 262144
_EPS = 1e-7
_ROWS = _N // 128
_BLK = 256
_STEPS = _ROWS // _BLK


def _rpn_loss_kernel(ts_ref, os_ref, td_ref, od_ref, out_ref,
                     bce_ref, val_ref, reg_ref, pos_ref):
    i = pl.program_id(0)

    ts = ts_ref[...]
    osc = os_ref[...]
    valid = (ts != -1.0).astype(jnp.float32)
    o = jnp.clip(osc, _EPS, 1.0 - _EPS)
    bce = -(ts * jnp.log(o) + (1.0 - ts) * jnp.log(1.0 - o))
    p_star = (ts > 0.0).astype(jnp.float32)

    diff = jnp.abs(od_ref[...] - td_ref[...])
    sl1 = jnp.where(diff < 1.0, 0.5 * diff * diff, diff - 0.5)
    a_y = jnp.sum(sl1, axis=1)

    @pl.when(i == 0)
    def _init():
        bce_ref[...] = bce * valid
        val_ref[...] = valid
        reg_ref[...] = a_y * p_star
        pos_ref[...] = p_star

    @pl.when(i > 0)
    def _accum():
        bce_ref[...] += bce * valid
        val_ref[...] += valid
        reg_ref[...] += a_y * p_star
        pos_ref[...] += p_star

    @pl.when(i == _STEPS - 1)
    def _finalize():
        cls_loss = jnp.sum(bce_ref[...]) / jnp.maximum(jnp.sum(val_ref[...]), 1.0)
        reg_loss = 10.0 * jnp.sum(reg_ref[...]) / jnp.maximum(_EPS, jnp.sum(pos_ref[...]))
        out_ref[0, 0] = cls_loss + reg_loss


def kernel(target_deltas, target_scores, output_deltas, output_scores):
    ts = target_scores.reshape(_ROWS, 128)
    osc = output_scores.reshape(_ROWS, 128)
    td = jnp.transpose(target_deltas.reshape(_ROWS, 128, 4), (0, 2, 1))
    od = jnp.transpose(output_deltas.reshape(_ROWS, 128, 4), (0, 2, 1))

    out = pl.pallas_call(
        _rpn_loss_kernel,
        grid=(_STEPS,),
        in_specs=[
            pl.BlockSpec((_BLK, 128), lambda i: (i, 0)),
            pl.BlockSpec((_BLK, 128), lambda i: (i, 0)),
            pl.BlockSpec((_BLK, 4, 128), lambda i: (i, 0, 0)),
            pl.BlockSpec((_BLK, 4, 128), lambda i: (i, 0, 0)),
        ],
        out_specs=pl.BlockSpec((1, 1), lambda i: (0, 0), memory_space=pltpu.SMEM),
        out_shape=jax.ShapeDtypeStruct((1, 1), jnp.float32),
        scratch_shapes=[
            pltpu.VMEM((_BLK, 128), jnp.float32),
            pltpu.VMEM((_BLK, 128), jnp.float32),
            pltpu.VMEM((_BLK, 128), jnp.float32),
            pltpu.VMEM((_BLK, 128), jnp.float32),
        ],
        compiler_params=pltpu.CompilerParams(
            dimension_semantics=("arbitrary",),
        ),
    )(ts, osc, td, od)
    return out[0, 0]


# speedup vs baseline: 36.0053x; 1.2551x over previous
import jax
import jax.numpy as jnp
from jax.experimental import pallas as pl
from jax.experimental.pallas import tpu as pltpu

_N = 262144
_EPS = 1e-7
_ROWS = _N // 128
_DROWS = 4 * _ROWS
_BLK = 256
_DBLK = 4 * _BLK
_STEPS = _ROWS // _BLK


def _rpn_loss_kernel(ts_ref, os_ref, td_ref, od_ref, out_ref,
                     bce_ref, val_ref, reg_ref, pos_ref):
    i = pl.program_id(0)

    ts = ts_ref[...]
    osc = os_ref[...]
    valid = (ts != -1.0).astype(jnp.float32)
    o = jnp.clip(osc, _EPS, 1.0 - _EPS)
    bce = -(ts * jnp.log(o) + (1.0 - ts) * jnp.log(1.0 - o))
    p_star = (ts > 0.0).astype(jnp.float32)

    diff = jnp.abs(od_ref[...] - td_ref[...])
    sl1 = jnp.where(diff < 1.0, 0.5 * diff * diff, diff - 0.5)
    mask = jnp.broadcast_to(p_star[:, None, :], (_BLK, 4, 128)).reshape(_DBLK, 128)

    @pl.when(i == 0)
    def _init():
        bce_ref[...] = bce * valid
        val_ref[...] = valid
        reg_ref[...] = sl1 * mask
        pos_ref[...] = p_star

    @pl.when(i > 0)
    def _accum():
        bce_ref[...] += bce * valid
        val_ref[...] += valid
        reg_ref[...] += sl1 * mask
        pos_ref[...] += p_star

    @pl.when(i == _STEPS - 1)
    def _finalize():
        cls_loss = jnp.sum(bce_ref[...]) / jnp.maximum(jnp.sum(val_ref[...]), 1.0)
        reg_loss = 10.0 * jnp.sum(reg_ref[...]) / jnp.maximum(_EPS, jnp.sum(pos_ref[...]))
        out_ref[0, 0] = cls_loss + reg_loss


def kernel(target_deltas, target_scores, output_deltas, output_scores):
    ts = target_scores.reshape(_ROWS, 128)
    osc = output_scores.reshape(_ROWS, 128)
    td = jnp.transpose(target_deltas.reshape(_ROWS, 128, 4), (0, 2, 1)).reshape(_DROWS, 128)
    od = jnp.transpose(output_deltas.reshape(_ROWS, 128, 4), (0, 2, 1)).reshape(_DROWS, 128)

    out = pl.pallas_call(
        _rpn_loss_kernel,
        grid=(_STEPS,),
        in_specs=[
            pl.BlockSpec((_BLK, 128), lambda i: (i, 0)),
            pl.BlockSpec((_BLK, 128), lambda i: (i, 0)),
            pl.BlockSpec((_DBLK, 128), lambda i: (i, 0)),
            pl.BlockSpec((_DBLK, 128), lambda i: (i, 0)),
        ],
        out_specs=pl.BlockSpec((1, 1), lambda i: (0, 0), memory_space=pltpu.SMEM),
        out_shape=jax.ShapeDtypeStruct((1, 1), jnp.float32),
        scratch_shapes=[
            pltpu.VMEM((_BLK, 128), jnp.float32),
            pltpu.VMEM((_BLK, 128), jnp.float32),
            pltpu.VMEM((_DBLK, 128), jnp.float32),
            pltpu.VMEM((_BLK, 128), jnp.float32),
        ],
        compiler_params=pltpu.CompilerParams(
            dimension_semantics=("arbitrary",),
        ),
    )(ts, osc, td, od)
    return out[0, 0]


# speedup vs baseline: 42.9031x; 1.1916x over previous
import jax
import jax.numpy as jnp
from jax.experimental import pallas as pl
from jax.experimental.pallas import tpu as pltpu

_N = 262144
_EPS = 1e-7
_ROWS = _N // 128
_DROWS = 4 * _ROWS
_BLK = 512
_DBLK = 4 * _BLK
_STEPS = _ROWS // _BLK


def _rpn_loss_kernel(ts_ref, os_ref, td_ref, od_ref, out_ref,
                     bce_ref, val_ref, reg_ref, pos_ref):
    i = pl.program_id(0)

    ts = ts_ref[...]
    osc = os_ref[...]
    valid = (ts != -1.0).astype(jnp.float32)
    pos = ts > 0.0
    p_star = pos.astype(jnp.float32)
    o = jnp.clip(osc, _EPS, 1.0 - _EPS)
    bce = -jnp.log(jnp.where(pos, o, 1.0 - o))

    d = jnp.abs(od_ref[...] - td_ref[...])
    m = jnp.minimum(d, 1.0)
    sl1 = m * (d - 0.5 * m)
    mask = jnp.broadcast_to(p_star[:, None, :], (_BLK, 4, 128)).reshape(_DBLK, 128)

    @pl.when(i == 0)
    def _init():
        bce_ref[...] = bce * valid
        val_ref[...] = valid
        reg_ref[...] = sl1 * mask
        pos_ref[...] = p_star

    @pl.when(i > 0)
    def _accum():
        bce_ref[...] += bce * valid
        val_ref[...] += valid
        reg_ref[...] += sl1 * mask
        pos_ref[...] += p_star

    @pl.when(i == _STEPS - 1)
    def _finalize():
        cls_loss = jnp.sum(bce_ref[...]) / jnp.maximum(jnp.sum(val_ref[...]), 1.0)
        reg_loss = 10.0 * jnp.sum(reg_ref[...]) / jnp.maximum(_EPS, jnp.sum(pos_ref[...]))
        out_ref[0, 0] = cls_loss + reg_loss


def kernel(target_deltas, target_scores, output_deltas, output_scores):
    ts = target_scores.reshape(_ROWS, 128)
    osc = output_scores.reshape(_ROWS, 128)
    td = jnp.transpose(target_deltas.reshape(_ROWS, 128, 4), (0, 2, 1)).reshape(_DROWS, 128)
    od = jnp.transpose(output_deltas.reshape(_ROWS, 128, 4), (0, 2, 1)).reshape(_DROWS, 128)

    out = pl.pallas_call(
        _rpn_loss_kernel,
        grid=(_STEPS,),
        in_specs=[
            pl.BlockSpec((_BLK, 128), lambda i: (i, 0)),
            pl.BlockSpec((_BLK, 128), lambda i: (i, 0)),
            pl.BlockSpec((_DBLK, 128), lambda i: (i, 0)),
            pl.BlockSpec((_DBLK, 128), lambda i: (i, 0)),
        ],
        out_specs=pl.BlockSpec((1, 1), lambda i: (0, 0), memory_space=pltpu.SMEM),
        out_shape=jax.ShapeDtypeStruct((1, 1), jnp.float32),
        scratch_shapes=[
            pltpu.VMEM((_BLK, 128), jnp.float32),
            pltpu.VMEM((_BLK, 128), jnp.float32),
            pltpu.VMEM((_DBLK, 128), jnp.float32),
            pltpu.VMEM((_BLK, 128), jnp.float32),
        ],
        compiler_params=pltpu.CompilerParams(
            dimension_semantics=("arbitrary",),
        ),
    )(ts, osc, td, od)
    return out[0, 0]
